# baseline (speedup 1.0000x reference)
import jax
import jax.numpy as jnp
from jax import lax
from jax.experimental import pallas as pl
from jax.experimental.pallas import tpu as pltpu
from jax.experimental.pallas import tpu_sc as plsc

NC = 2
NS = 16
L = 16
D_MODEL = 1024
CHUNK = 16
NBUF = 4
REFILL_LAG = 2
CB = 4

SC_ROWS = 22528
TC_BLK = 512


def _sc_embed_add(n_tokens):
    nw = NC * NS
    tok_per_w = n_tokens // nw
    n_chunks = tok_per_w // CHUNK
    mesh = plsc.VectorSubcoreMesh(core_axis_name="c", subcore_axis_name="s")

    def body(x_hbm, idx_hbm, table_hbm, out_hbm,
             table_v, idx_v, bufs, in_sems, out_sems):
        wid = lax.axis_index("s") * NC + lax.axis_index("c")
        base = wid * tok_per_w
        pltpu.sync_copy(table_hbm, table_v)
        pltpu.sync_copy(idx_hbm.at[pl.ds(base, tok_per_w)], idx_v)

        for b in range(NBUF):
            pltpu.async_copy(
                x_hbm.at[pl.ds(base + b * CHUNK, CHUNK)], bufs.at[b],
                in_sems.at[b])

        def compute_chunk(ci, b):
            buf = bufs.at[b]

            @plsc.parallel_loop(0, D_MODEL // (CB * L))
            def cb_body(cb):
                col0 = cb * CB * L
                rows = [[table_v[q, pl.ds(col0 + k * L, L)]
                         for k in range(CB)] for q in range(4)]
                iv = idx_v[pl.ds(ci * CHUNK, L)]
                for t in range(L):
                    s = iv[t]
                    m_od = (s & 1) == 1
                    m_hi = s >= 2
                    for k in range(CB):
                        hi = jnp.where(m_od, rows[3][k], rows[2][k])
                        lo = jnp.where(m_od, rows[1][k], rows[0][k])
                        e = jnp.where(m_hi, hi, lo)
                        plsc.addupdate(
                            buf.at[t, pl.ds(col0 + k * L, L)], e)

        def outer(r, _):
            for b in range(NBUF):
                ci = r * NBUF + b
                tok0 = base + ci * CHUNK
                pltpu.make_async_copy(
                    x_hbm.at[pl.ds(tok0, CHUNK)], bufs.at[b],
                    in_sems.at[b]).wait()
                compute_chunk(ci, b)
                pltpu.async_copy(
                    bufs.at[b], out_hbm.at[pl.ds(tok0, CHUNK)],
                    out_sems.at[b])

                cj = ci + NBUF - REFILL_LAG
                bj = (b + NBUF - REFILL_LAG) % NBUF

                @pl.when(jnp.logical_and(ci >= REFILL_LAG,
                                         cj < n_chunks))
                def _():
                    tokj = base + cj * CHUNK
                    pltpu.make_async_copy(
                        bufs.at[bj],
                        out_hbm.at[pl.ds(tokj - NBUF * CHUNK, CHUNK)],
                        out_sems.at[bj]).wait()
                    pltpu.async_copy(
                        x_hbm.at[pl.ds(tokj, CHUNK)], bufs.at[bj],
                        in_sems.at[bj])
            return 0

        lax.fori_loop(0, n_chunks // NBUF, outer, 0)

        for b in range(NBUF):
            pltpu.make_async_copy(
                bufs.at[b],
                out_hbm.at[pl.ds(base + (n_chunks - NBUF + b) * CHUNK, CHUNK)],
                out_sems.at[b]).wait()

    return pl.kernel(
        body,
        out_type=jax.ShapeDtypeStruct((n_tokens, D_MODEL), jnp.float32),
        mesh=mesh,
        scratch_types=[
            pltpu.VMEM((4, D_MODEL), jnp.float32),
            pltpu.VMEM((tok_per_w,), jnp.int32),
            pltpu.VMEM((NBUF, CHUNK, D_MODEL), jnp.float32),
            pltpu.SemaphoreType.DMA((NBUF,)),
            pltpu.SemaphoreType.DMA((NBUF,)),
        ],
    )


def _tc_body(idx_ref, table_ref, x_ref, o_ref):
    ib = idx_ref[0]
    t = table_ref[...]
    sel_od = ib % 2 == 1
    hi = jnp.where(sel_od, t[3][None, :], t[2][None, :])
    lo = jnp.where(sel_od, t[1][None, :], t[0][None, :])
    e = jnp.where(ib >= 2, hi, lo)
    o_ref[...] = x_ref[...] + e


def _tc_embed_add(n_rows):
    grid = (n_rows // TC_BLK,)
    return pl.pallas_call(
        _tc_body,
        grid=grid,
        in_specs=[
            pl.BlockSpec((1, TC_BLK, 1), lambda i: (i, 0, 0)),
            pl.BlockSpec((4, D_MODEL), lambda i: (0, 0)),
            pl.BlockSpec((TC_BLK, D_MODEL), lambda i: (i, 0)),
        ],
        out_specs=pl.BlockSpec((TC_BLK, D_MODEL), lambda i: (i, 0)),
        out_shape=jax.ShapeDtypeStruct((n_rows, D_MODEL), jnp.float32),
    )


def kernel(x, lirads, table):
    b, s, d = x.shape
    n = b * s
    xf = x.reshape(n, d)
    idx = lirads.reshape(n).astype(jnp.int32)

    n_sc = SC_ROWS
    n_tc = n - n_sc
    sc_out = _sc_embed_add(n_sc)(xf[:n_sc], idx[:n_sc], table)
    idx_tc = idx[n_sc:].reshape(n_tc // TC_BLK, TC_BLK, 1)
    tc_out = _tc_embed_add(n_tc)(idx_tc, table, xf[n_sc:])
    out = jnp.concatenate([sc_out, tc_out], axis=0)
    return out.reshape(b, s, d)


# speedup vs baseline: 2.4512x; 2.4512x over previous
import jax
import jax.numpy as jnp
from jax import lax
from jax.experimental import pallas as pl
from jax.experimental.pallas import tpu as pltpu
from jax.experimental.pallas import tpu_sc as plsc

NC = 2
NS = 16
L = 16
D_MODEL = 1024
CHUNK = 16
NBUF = 4
SPB = 2
CB = 4


def _sc_embed_add(n_tokens):
    nw = NC * NS
    tok_per_w = n_tokens // nw
    n_chunks = tok_per_w // CHUNK
    mesh = plsc.VectorSubcoreMesh(core_axis_name="c", subcore_axis_name="s")

    def body(x_hbm, idx_hbm, table_hbm, out_hbm,
             table_v, idx_v, bufs, sp, in_sems, loc_sems, out_sems):
        wid = lax.axis_index("s") * NC + lax.axis_index("c")
        sid = lax.axis_index("s")
        base = wid * tok_per_w
        pltpu.sync_copy(table_hbm, table_v)
        pltpu.sync_copy(idx_hbm.at[pl.ds(base, tok_per_w)], idx_v)

        for b in range(NBUF):
            pltpu.async_copy(
                x_hbm.at[pl.ds(base + b * CHUNK, CHUNK)], bufs.at[b],
                in_sems.at[b])

        def compute_chunk(ci, b):
            buf = bufs.at[b]

            @plsc.parallel_loop(0, D_MODEL // (CB * L))
            def cb_body(cb):
                col0 = cb * CB * L
                rows = [[table_v[q, pl.ds(col0 + k * L, L)]
                         for k in range(CB)] for q in range(4)]
                iv = idx_v[pl.ds(ci * CHUNK, L)]
                for t in range(L):
                    s = iv[t]
                    m_od = (s & 1) == 1
                    m_hi = s >= 2
                    for k in range(CB):
                        hi = jnp.where(m_od, rows[3][k], rows[2][k])
                        lo = jnp.where(m_od, rows[1][k], rows[0][k])
                        e = jnp.where(m_hi, hi, lo)
                        plsc.addupdate(
                            buf.at[t, pl.ds(col0 + k * L, L)], e)

        def outer(r, _):
            for b in range(NBUF):
                ci = r * NBUF + b
                tok0 = base + ci * CHUNK
                bp = (b + NBUF - 1) % NBUF
                sb = b % SPB
                sbp = (b + SPB - 1) % SPB

                @pl.when(ci >= 1)
                def _():
                    cp = ci - 1
                    tokp = base + cp * CHUNK
                    pltpu.make_async_copy(
                        bufs.at[bp], sp.at[sid, sbp], loc_sems.at[bp]).wait()
                    pltpu.async_copy(
                        sp.at[sid, sbp], out_hbm.at[pl.ds(tokp, CHUNK)],
                        out_sems.at[sbp])

                    @pl.when(cp + NBUF < n_chunks)
                    def _():
                        pltpu.async_copy(
                            x_hbm.at[pl.ds(tokp + NBUF * CHUNK, CHUNK)],
                            bufs.at[bp], in_sems.at[bp])

                pltpu.make_async_copy(
                    x_hbm.at[pl.ds(tok0, CHUNK)], bufs.at[b],
                    in_sems.at[b]).wait()
                compute_chunk(ci, b)

                @pl.when(ci >= SPB)
                def _():
                    pltpu.make_async_copy(
                        sp.at[sid, sb],
                        out_hbm.at[pl.ds(tok0 - SPB * CHUNK, CHUNK)],
                        out_sems.at[sb]).wait()
                pltpu.async_copy(bufs.at[b], sp.at[sid, sb], loc_sems.at[b])
            return 0

        lax.fori_loop(0, n_chunks // NBUF, outer, 0)

        bl = (n_chunks - 1) % NBUF
        sl = (n_chunks - 1) % SPB
        pltpu.make_async_copy(
            bufs.at[bl], sp.at[sid, sl], loc_sems.at[bl]).wait()
        pltpu.async_copy(
            sp.at[sid, sl],
            out_hbm.at[pl.ds(base + (n_chunks - 1) * CHUNK, CHUNK)],
            out_sems.at[sl])
        for k in range(SPB):
            c = n_chunks - SPB + k
            pltpu.make_async_copy(
                sp.at[sid, c % SPB],
                out_hbm.at[pl.ds(base + c * CHUNK, CHUNK)],
                out_sems.at[c % SPB]).wait()

    return pl.kernel(
        body,
        out_type=jax.ShapeDtypeStruct((n_tokens, D_MODEL), jnp.float32),
        mesh=mesh,
        scratch_types=[
            pltpu.VMEM((4, D_MODEL), jnp.float32),
            pltpu.VMEM((tok_per_w,), jnp.int32),
            pltpu.VMEM((NBUF, CHUNK, D_MODEL), jnp.float32),
            pltpu.VMEM_SHARED((NS, SPB, CHUNK, D_MODEL), jnp.float32),
            pltpu.SemaphoreType.DMA((NBUF,)),
            pltpu.SemaphoreType.DMA((NBUF,)),
            pltpu.SemaphoreType.DMA((SPB,)),
        ],
    )


def kernel(x, lirads, table):
    b, s, d = x.shape
    n = b * s
    xf = x.reshape(n, d)
    idx = lirads.reshape(n).astype(jnp.int32)
    out = _sc_embed_add(n)(xf, idx, table)
    return out.reshape(b, s, d)


# speedup vs baseline: 2.5218x; 1.0288x over previous
import jax
import jax.numpy as jnp
from jax import lax
from jax.experimental import pallas as pl
from jax.experimental.pallas import tpu as pltpu
from jax.experimental.pallas import tpu_sc as plsc

NC = 2
NS = 16
L = 16
D_MODEL = 1024
CHUNK = 16
NBUF = 4
REFILL_LAG = 2
CB = 4


def _sc_embed_add(n_tokens):
    nw = NC * NS
    tok_per_w = n_tokens // nw
    n_chunks = tok_per_w // CHUNK
    mesh = plsc.VectorSubcoreMesh(core_axis_name="c", subcore_axis_name="s")

    def body(x_hbm, idx_hbm, table_hbm, out_hbm,
             table_v, idx_v, bufs, in_sems, out_sems):
        wid = lax.axis_index("s") * NC + lax.axis_index("c")
        base = wid * tok_per_w
        pltpu.sync_copy(table_hbm, table_v)
        pltpu.sync_copy(idx_hbm.at[pl.ds(base, tok_per_w)], idx_v)

        for b in range(NBUF):
            pltpu.async_copy(
                x_hbm.at[pl.ds(base + b * CHUNK, CHUNK)], bufs.at[b],
                in_sems.at[b])

        def compute_chunk(ci, b):
            buf = bufs.at[b]

            @plsc.parallel_loop(0, D_MODEL // (CB * L))
            def cb_body(cb):
                col0 = cb * CB * L
                rows = [[table_v[q, pl.ds(col0 + k * L, L)]
                         for k in range(CB)] for q in range(4)]
                iv = idx_v[pl.ds(ci * CHUNK, L)]
                for t in range(L):
                    s = iv[t]
                    m_od = (s & 1) == 1
                    m_hi = s >= 2
                    for k in range(CB):
                        hi = jnp.where(m_od, rows[3][k], rows[2][k])
                        lo = jnp.where(m_od, rows[1][k], rows[0][k])
                        e = jnp.where(m_hi, hi, lo)
                        plsc.addupdate(
                            buf.at[t, pl.ds(col0 + k * L, L)], e)

        def outer(r, _):
            for b in range(NBUF):
                ci = r * NBUF + b
                tok0 = base + ci * CHUNK
                pltpu.make_async_copy(
                    x_hbm.at[pl.ds(tok0, CHUNK)], bufs.at[b],
                    in_sems.at[b]).wait()
                compute_chunk(ci, b)
                pltpu.async_copy(
                    bufs.at[b], out_hbm.at[pl.ds(tok0, CHUNK)],
                    out_sems.at[b])

                cj = ci + NBUF - REFILL_LAG
                bj = (b + NBUF - REFILL_LAG) % NBUF

                @pl.when(jnp.logical_and(ci >= REFILL_LAG,
                                         cj < n_chunks))
                def _():
                    tokj = base + cj * CHUNK
                    pltpu.make_async_copy(
                        bufs.at[bj],
                        out_hbm.at[pl.ds(tokj - NBUF * CHUNK, CHUNK)],
                        out_sems.at[bj]).wait()
                    pltpu.async_copy(
                        x_hbm.at[pl.ds(tokj, CHUNK)], bufs.at[bj],
                        in_sems.at[bj])
            return 0

        lax.fori_loop(0, n_chunks // NBUF, outer, 0)

        for b in range(NBUF):
            pltpu.make_async_copy(
                bufs.at[b],
                out_hbm.at[pl.ds(base + (n_chunks - NBUF + b) * CHUNK, CHUNK)],
                out_sems.at[b]).wait()

    return pl.kernel(
        body,
        out_type=jax.ShapeDtypeStruct((n_tokens, D_MODEL), jnp.float32),
        mesh=mesh,
        scratch_types=[
            pltpu.VMEM((4, D_MODEL), jnp.float32),
            pltpu.VMEM((tok_per_w,), jnp.int32),
            pltpu.VMEM((NBUF, CHUNK, D_MODEL), jnp.float32),
            pltpu.SemaphoreType.DMA((NBUF,)),
            pltpu.SemaphoreType.DMA((NBUF,)),
        ],
    )


def kernel(x, lirads, table):
    b, s, d = x.shape
    n = b * s
    xf = x.reshape(n, d)
    idx = lirads.reshape(n).astype(jnp.int32)
    out = _sc_embed_add(n)(xf, idx, table)
    return out.reshape(b, s, d)
